# baseline (speedup 1.0000x reference)
import functools

import jax
import jax.numpy as jnp
from jax import lax
from jax.experimental import pallas as pl
from jax.experimental.pallas import tpu as pltpu
from jax.experimental.pallas import tpu_sc as plsc

HASH_BITS = 20
HASH_SIZE = 2 ** HASH_BITS
DIM = 64
N = 16384
PRIME_X = 73856093
PRIME_Y = 19349663

_INFO = plsc.get_sparse_core_info()
_NC = _INFO.num_cores
_NS = _INFO.num_subcores
_NW = _NC * _NS
_L = _INFO.num_lanes
_BPW = N // _NW
_GCHUNK = 128
_NG = _BPW // _GCHUNK


def _sc_body(pos_hbm, table_hbm, out_hbm, pos_v, idx_v, rows_v, sem):
    c = lax.axis_index("c")
    s = lax.axis_index("s")
    wid = s * _NC + c
    base = wid * _BPW

    pltpu.sync_copy(pos_hbm.at[pl.ds(2 * base, 2 * _BPW)], pos_v)

    lane = lax.iota(jnp.int32, _L)

    def hash_of(p):
        t = p.astype(jnp.int32)
        return t - (t.astype(jnp.float32) > p).astype(jnp.int32)

    for i in range(_BPW // _L):
        gx = lane * 2 + (2 * _L * i)
        px = plsc.load_gather(pos_v, [gx])
        py = plsc.load_gather(pos_v, [gx + 1])
        ix = hash_of(px)
        iy = hash_of(py)
        h = ((ix * PRIME_X) ^ (iy * PRIME_Y)) & (HASH_SIZE - 1)
        idx_v[i // (_GCHUNK // _L), pl.ds((i % (_GCHUNK // _L)) * _L, _L)] = h

    copies = [
        pltpu.async_copy(
            table_hbm.at[idx_v.at[jnp.int32(j)]],
            rows_v.at[pl.ds(j * _GCHUNK, _GCHUNK)],
            sem,
        )
        for j in range(_NG)
    ]
    for cp in copies:
        cp.wait()

    pltpu.sync_copy(rows_v, out_hbm.at[pl.ds(base, _BPW)])


@jax.jit
def _hash_grid_lookup(pos_flat, table):
    mesh = plsc.VectorSubcoreMesh(core_axis_name="c", subcore_axis_name="s")
    k = functools.partial(
        pl.kernel,
        mesh=mesh,
        compiler_params=pltpu.CompilerParams(
            needs_layout_passes=False, use_tc_tiling_on_sc=False
        ),
        out_type=jax.ShapeDtypeStruct((N, DIM), jnp.float32),
        scratch_types=[
            pltpu.VMEM((2 * _BPW,), jnp.float32),
            pltpu.VMEM((_NG, _GCHUNK), jnp.int32),
            pltpu.VMEM((_BPW, DIM), jnp.float32),
            pltpu.SemaphoreType.DMA,
        ],
    )(_sc_body)
    return k(pos_flat, table)


def kernel(positions, table):
    pos_flat = positions.reshape(2 * N)
    return _hash_grid_lookup(pos_flat, table)
